# baseline (speedup 1.0000x reference)
import functools

import jax
import jax.numpy as jnp
from jax.experimental import pallas as pl
from jax.experimental.pallas import tpu as pltpu

_NUM_SEGMENTS = 128


def _body(bsm_ref, x_ref, b_ref, wa_ref, ba_ref, cx_ref, wo_ref, bo_ref,
          out_ref, m_ref, z_ref, acc_ref, mrow_ref):
    i = pl.program_id(0)
    num_blocks = pl.num_programs(0)
    neg_inf = jnp.float32(-jnp.inf)
    num_seg = m_ref.shape[0]
    k = x_ref.shape[0]
    ws = num_seg if num_seg < 32 else 32
    nw = num_seg // ws

    @pl.when(i == 0)
    def _init():
        m_ref[...] = jnp.full(m_ref.shape, neg_inf, jnp.float32)
        z_ref[...] = jnp.zeros(z_ref.shape, jnp.float32)
        acc_ref[...] = jnp.zeros(acc_ref.shape, jnp.float32)

    xb16 = x_ref[...].astype(jnp.bfloat16)
    logits = jax.lax.dot_general(
        wa_ref[...].astype(jnp.bfloat16), xb16, (((1,), (1,)), ((), ())),
        preferred_element_type=jnp.float32) + ba_ref[...]
    t = jnp.tanh(logits)
    s = jnp.sum(t * cx_ref[...], axis=0, keepdims=True)

    bv = b_ref[0]
    lo = bsm_ref[i * k]
    hi = bsm_ref[i * k + (k - 1)]

    mrow_ref[...] = jnp.zeros(mrow_ref.shape, jnp.float32)
    iota = jax.lax.broadcasted_iota(jnp.int32, (ws, 1), 0)

    for w in range(nw):
        @pl.when((lo <= w * ws + (ws - 1)) & (hi >= w * ws))
        def _stats(w=w):
            sl = slice(w * ws, (w + 1) * ws)
            onehot = (iota + w * ws) == bv
            m_blk = jnp.max(jnp.where(onehot, s, neg_inf), axis=1,
                            keepdims=True)
            m_old = m_ref[sl, :]
            m_new = jnp.maximum(m_old, m_blk).astype(
                jnp.bfloat16).astype(jnp.float32)
            scale = jnp.where(m_new == neg_inf, 1.0,
                              jnp.exp(m_old - m_new))
            m_safe = jnp.where(m_new == neg_inf, 0.0, m_new)
            mrow_ref[...] += jnp.sum(
                jnp.where(onehot, m_safe, 0.0), axis=0,
                keepdims=True)
            m_ref[sl, :] = m_new
            z_ref[sl, :] = z_ref[sl, :] * scale
            acc_ref[sl, :] = acc_ref[sl, :] * scale

    e = jnp.exp(s - mrow_ref[...])
    ones = jnp.ones((k, 1), jnp.bfloat16)

    for w in range(nw):
        @pl.when((lo <= w * ws + (ws - 1)) & (hi >= w * ws))
        def _pool(w=w):
            sl = slice(w * ws, (w + 1) * ws)
            onehot = (iota + w * ws) == bv
            w16 = jnp.where(onehot, e, 0.0).astype(jnp.bfloat16)
            z_ref[sl, :] += jax.lax.dot_general(
                w16, ones, (((1,), (0,)), ((), ())),
                preferred_element_type=jnp.float32)
            acc_ref[sl, :] += jax.lax.dot_general(
                w16, xb16, (((1,), (0,)), ((), ())),
                preferred_element_type=jnp.float32)

    @pl.when(i == num_blocks - 1)
    def _finish():
        rep = acc_ref[...] / (z_ref[...] + 1e-8)
        out_ref[...] = jax.lax.dot_general(
            rep.astype(jnp.bfloat16), wo_ref[...].astype(jnp.bfloat16),
            (((1,), (1,)), ((), ())),
            preferred_element_type=jnp.float32) + bo_ref[...]


def _pick_block(n):
    for k in range(min(n, 10000), 7, -1):
        if n % k == 0 and k % 8 == 0:
            return k
    return None


@functools.partial(jax.jit, static_argnames=("num_segments", "interpret"))
def _pooled_attention(x, batch, W_att, b_att, context, W_out, b_out,
                      num_segments=_NUM_SEGMENTS, interpret=False):
    n, d = x.shape
    a = W_att.shape[0]
    k = _pick_block(n)
    if k is None:
        k = min(2048, 8 * ((n + 7) // 8))
        n_pad = ((n + k - 1) // k) * k
        x = jnp.pad(x, ((0, n_pad - n), (0, 0)))
        batch = jnp.pad(batch, (0, n_pad - n),
                        constant_values=num_segments)
        n = n_pad
    g = n // k

    batch3 = batch.reshape(g, 1, k)
    ba2 = b_att.reshape(a, 1)
    cx2 = context.reshape(a, 1)
    bo2 = b_out.reshape(1, d)

    grid_spec = pltpu.PrefetchScalarGridSpec(
        num_scalar_prefetch=1,
        grid=(g,),
        in_specs=[
            pl.BlockSpec((k, d), lambda i, bsm: (i, 0)),
            pl.BlockSpec((1, 1, k), lambda i, bsm: (i, 0, 0)),
            pl.BlockSpec((a, d), lambda i, bsm: (0, 0)),
            pl.BlockSpec((a, 1), lambda i, bsm: (0, 0)),
            pl.BlockSpec((a, 1), lambda i, bsm: (0, 0)),
            pl.BlockSpec((d, d), lambda i, bsm: (0, 0)),
            pl.BlockSpec((1, d), lambda i, bsm: (0, 0)),
        ],
        out_specs=pl.BlockSpec((num_segments, d), lambda i, bsm: (0, 0)),
        scratch_shapes=[
            pltpu.VMEM((num_segments, 1), jnp.float32),
            pltpu.VMEM((num_segments, 1), jnp.float32),
            pltpu.VMEM((num_segments, d), jnp.float32),
            pltpu.VMEM((1, k), jnp.float32),
        ],
    )
    out = pl.pallas_call(
        _body,
        grid_spec=grid_spec,
        out_shape=jax.ShapeDtypeStruct((num_segments, d), jnp.float32),
        compiler_params=pltpu.CompilerParams(
            dimension_semantics=("arbitrary",)),
        interpret=interpret,
    )(batch, x, batch3, W_att, ba2, cx2, W_out, bo2)
    return out


def kernel(x, batch, W_att, b_att, context, W_out, b_out):
    return _pooled_attention(x, batch, W_att, b_att, context, W_out, b_out)


# speedup vs baseline: 1.0623x; 1.0623x over previous
import functools

import jax
import jax.numpy as jnp
from jax.experimental import pallas as pl
from jax.experimental.pallas import tpu as pltpu

_NUM_SEGMENTS = 128


def _body(x_ref, b_ref, wa_ref, ba_ref, cx_ref, wo_ref, bo_ref, out_ref,
          m_ref, z_ref, acc_ref, xs_ref, ws_ref):
    i = pl.program_id(0)
    num_blocks = pl.num_programs(0)
    neg_inf = jnp.float32(-jnp.inf)
    num_seg = m_ref.shape[0]
    k = x_ref.shape[0]
    j = i % 2

    @pl.when(i == 0)
    def _init():
        m_ref[...] = jnp.full(m_ref.shape, neg_inf, jnp.float32)
        z_ref[...] = jnp.zeros(z_ref.shape, jnp.float32)
        acc_ref[...] = jnp.zeros(acc_ref.shape, jnp.float32)

    ones = jnp.ones((k, 1), jnp.bfloat16)

    @pl.when(i > 0)
    def _flush():
        pj = (i - 1) % 2
        xp = xs_ref[pl.ds(pj * k, k), :]
        wp = ws_ref[pl.ds(pj * num_seg, num_seg), :]
        z_ref[...] += jax.lax.dot_general(
            wp, ones, (((1,), (0,)), ((), ())),
            preferred_element_type=jnp.float32)
        acc_ref[...] += jax.lax.dot_general(
            wp, xp, (((1,), (0,)), ((), ())),
            preferred_element_type=jnp.float32)

    xb16 = x_ref[...].astype(jnp.bfloat16)
    xs_ref[pl.ds(j * k, k), :] = xb16
    logits = jax.lax.dot_general(
        wa_ref[...].astype(jnp.bfloat16), xb16, (((1,), (1,)), ((), ())),
        preferred_element_type=jnp.float32) + ba_ref[...]
    t = jnp.tanh(logits)
    s = jnp.sum(t * cx_ref[...], axis=0, keepdims=True)

    bv = b_ref[0]
    seg = jax.lax.broadcasted_iota(jnp.int32, (num_seg, 1), 0)
    onehot = seg == bv
    oh16 = onehot.astype(jnp.bfloat16)

    m_blk = jnp.max(jnp.where(onehot, s, neg_inf), axis=1, keepdims=True)
    m_old = m_ref[...]
    m_new = jnp.maximum(m_old, m_blk).astype(jnp.bfloat16).astype(jnp.float32)
    scale = jnp.where(m_new == neg_inf, 1.0, jnp.exp(m_old - m_new))

    m_safe16 = jnp.where(m_new == neg_inf, 0.0, m_new).astype(jnp.bfloat16)
    m_row = jax.lax.dot_general(
        m_safe16, oh16, (((0,), (0,)), ((), ())),
        preferred_element_type=jnp.float32)
    e16 = jnp.exp(s - m_row).astype(jnp.bfloat16)
    w16 = oh16 * e16

    z_ref[...] = z_ref[...] * scale
    acc_ref[...] = acc_ref[...] * scale
    m_ref[...] = m_new
    ws_ref[pl.ds(j * num_seg, num_seg), :] = w16

    @pl.when(i == num_blocks - 1)
    def _finish():
        z = z_ref[...] + jax.lax.dot_general(
            w16, ones, (((1,), (0,)), ((), ())),
            preferred_element_type=jnp.float32)
        acc = acc_ref[...] + jax.lax.dot_general(
            w16, xb16, (((1,), (0,)), ((), ())),
            preferred_element_type=jnp.float32)
        rep = acc / (z + 1e-8)
        out_ref[...] = jax.lax.dot_general(
            rep.astype(jnp.bfloat16), wo_ref[...].astype(jnp.bfloat16),
            (((1,), (1,)), ((), ())),
            preferred_element_type=jnp.float32) + bo_ref[...]


def _pick_block(n):
    for k in range(min(n, 5000), 7, -1):
        if n % k == 0 and k % 8 == 0:
            return k
    return None


@functools.partial(jax.jit, static_argnames=("num_segments", "interpret"))
def _pooled_attention(x, batch, W_att, b_att, context, W_out, b_out,
                      num_segments=_NUM_SEGMENTS, interpret=False):
    n, d = x.shape
    a = W_att.shape[0]
    k = _pick_block(n)
    if k is None:
        k = min(2048, 8 * ((n + 7) // 8))
        n_pad = ((n + k - 1) // k) * k
        x = jnp.pad(x, ((0, n_pad - n), (0, 0)))
        batch = jnp.pad(batch, (0, n_pad - n), constant_values=-1)
        n = n_pad
    g = n // k

    batch3 = batch.reshape(g, 1, k)
    ba2 = b_att.reshape(a, 1)
    cx2 = context.reshape(a, 1)
    bo2 = b_out.reshape(1, d)

    out = pl.pallas_call(
        _body,
        grid=(g,),
        in_specs=[
            pl.BlockSpec((k, d), lambda i: (i, 0)),
            pl.BlockSpec((1, 1, k), lambda i: (i, 0, 0)),
            pl.BlockSpec((a, d), lambda i: (0, 0)),
            pl.BlockSpec((a, 1), lambda i: (0, 0)),
            pl.BlockSpec((a, 1), lambda i: (0, 0)),
            pl.BlockSpec((d, d), lambda i: (0, 0)),
            pl.BlockSpec((1, d), lambda i: (0, 0)),
        ],
        out_specs=pl.BlockSpec((num_segments, d), lambda i: (0, 0)),
        out_shape=jax.ShapeDtypeStruct((num_segments, d), jnp.float32),
        scratch_shapes=[
            pltpu.VMEM((num_segments, 1), jnp.float32),
            pltpu.VMEM((num_segments, 1), jnp.float32),
            pltpu.VMEM((num_segments, d), jnp.float32),
            pltpu.VMEM((2 * k, d), jnp.bfloat16),
            pltpu.VMEM((2 * num_segments, k), jnp.bfloat16),
        ],
        compiler_params=pltpu.CompilerParams(
            dimension_semantics=("arbitrary",)),
        interpret=interpret,
    )(x, batch3, W_att, ba2, cx2, W_out, bo2)
    return out


def kernel(x, batch, W_att, b_att, context, W_out, b_out):
    return _pooled_attention(x, batch, W_att, b_att, context, W_out, b_out)


# speedup vs baseline: 1.1257x; 1.0597x over previous
import functools

import jax
import jax.numpy as jnp
from jax.experimental import pallas as pl
from jax.experimental.pallas import tpu as pltpu

_NUM_SEGMENTS = 128


def _body(x_ref, b_ref, wa_ref, ba_ref, cx_ref, wo_ref, bo_ref, out_ref,
          m_ref, z_ref, acc_ref):
    i = pl.program_id(0)
    num_blocks = pl.num_programs(0)
    neg_inf = jnp.float32(-jnp.inf)
    num_seg = m_ref.shape[0]
    k = x_ref.shape[0]

    @pl.when(i == 0)
    def _init():
        m_ref[...] = jnp.full(m_ref.shape, neg_inf, jnp.float32)
        z_ref[...] = jnp.zeros(z_ref.shape, jnp.float32)
        acc_ref[...] = jnp.zeros(acc_ref.shape, jnp.float32)

    xb16 = x_ref[...].astype(jnp.bfloat16)
    logits = jax.lax.dot_general(
        wa_ref[...].astype(jnp.bfloat16), xb16, (((1,), (1,)), ((), ())),
        preferred_element_type=jnp.float32) + ba_ref[...]
    t16 = jnp.tanh(logits).astype(jnp.bfloat16)
    s = jax.lax.dot_general(
        cx_ref[...].astype(jnp.bfloat16), t16, (((0,), (0,)), ((), ())),
        preferred_element_type=jnp.float32)

    bv = b_ref[0]
    seg = jax.lax.broadcasted_iota(jnp.int32, (num_seg, 1), 0)
    onehot = seg == bv
    oh16 = onehot.astype(jnp.bfloat16)

    m_blk = jnp.max(jnp.where(onehot, s, neg_inf), axis=1, keepdims=True)
    m_old = m_ref[...]
    m_new = jnp.maximum(m_old, m_blk).astype(jnp.bfloat16).astype(jnp.float32)
    scale = jnp.where(m_new == neg_inf, 1.0, jnp.exp(m_old - m_new))

    m_safe16 = jnp.where(m_new == neg_inf, 0.0, m_new).astype(jnp.bfloat16)
    m_row = jax.lax.dot_general(
        m_safe16, oh16, (((0,), (0,)), ((), ())),
        preferred_element_type=jnp.float32)
    e16 = jnp.exp(s - m_row).astype(jnp.bfloat16)
    w16 = oh16 * e16

    ones = jnp.ones((k, 1), jnp.bfloat16)
    z_blk = jax.lax.dot_general(
        w16, ones, (((1,), (0,)), ((), ())),
        preferred_element_type=jnp.float32)
    z_ref[...] = z_ref[...] * scale + z_blk
    acc_ref[...] = acc_ref[...] * scale + jax.lax.dot_general(
        w16, xb16, (((1,), (0,)), ((), ())),
        preferred_element_type=jnp.float32)
    m_ref[...] = m_new

    @pl.when(i == num_blocks - 1)
    def _finish():
        rep = acc_ref[...] / (z_ref[...] + 1e-8)
        out_ref[...] = jax.lax.dot_general(
            rep.astype(jnp.bfloat16), wo_ref[...].astype(jnp.bfloat16),
            (((1,), (1,)), ((), ())),
            preferred_element_type=jnp.float32) + bo_ref[...]


def _pick_block(n):
    for k in range(min(n, 5000), 7, -1):
        if n % k == 0 and k % 8 == 0:
            return k
    return None


@functools.partial(jax.jit, static_argnames=("num_segments", "interpret"))
def _pooled_attention(x, batch, W_att, b_att, context, W_out, b_out,
                      num_segments=_NUM_SEGMENTS, interpret=False):
    n, d = x.shape
    a = W_att.shape[0]
    k = _pick_block(n)
    if k is None:
        k = min(2048, 8 * ((n + 7) // 8))
        n_pad = ((n + k - 1) // k) * k
        x = jnp.pad(x, ((0, n_pad - n), (0, 0)))
        batch = jnp.pad(batch, (0, n_pad - n), constant_values=-1)
        n = n_pad
    g = n // k

    batch3 = batch.reshape(g, 1, k)
    ba2 = b_att.reshape(a, 1)
    cx2 = context.reshape(a, 1)
    bo2 = b_out.reshape(1, d)

    out = pl.pallas_call(
        _body,
        grid=(g,),
        in_specs=[
            pl.BlockSpec((k, d), lambda i: (i, 0)),
            pl.BlockSpec((1, 1, k), lambda i: (i, 0, 0)),
            pl.BlockSpec((a, d), lambda i: (0, 0)),
            pl.BlockSpec((a, 1), lambda i: (0, 0)),
            pl.BlockSpec((a, 1), lambda i: (0, 0)),
            pl.BlockSpec((d, d), lambda i: (0, 0)),
            pl.BlockSpec((1, d), lambda i: (0, 0)),
        ],
        out_specs=pl.BlockSpec((num_segments, d), lambda i: (0, 0)),
        out_shape=jax.ShapeDtypeStruct((num_segments, d), jnp.float32),
        scratch_shapes=[
            pltpu.VMEM((num_segments, 1), jnp.float32),
            pltpu.VMEM((num_segments, 1), jnp.float32),
            pltpu.VMEM((num_segments, d), jnp.float32),
        ],
        compiler_params=pltpu.CompilerParams(
            dimension_semantics=("arbitrary",)),
        interpret=interpret,
    )(x, batch3, W_att, ba2, cx2, W_out, bo2)
    return out


def kernel(x, batch, W_att, b_att, context, W_out, b_out):
    return _pooled_attention(x, batch, W_att, b_att, context, W_out, b_out)


# speedup vs baseline: 1.1638x; 1.0338x over previous
import functools

import jax
import jax.numpy as jnp
from jax.experimental import pallas as pl
from jax.experimental.pallas import tpu as pltpu

_NUM_SEGMENTS = 128


def _body(x_ref, b_ref, wa_ref, ba_ref, cx_ref, wo_ref, bo_ref, out_ref,
          m_ref, z_ref, acc_ref):
    i = pl.program_id(0)
    num_blocks = pl.num_programs(0)
    neg_inf = jnp.float32(-jnp.inf)
    num_seg = m_ref.shape[0]
    k = x_ref.shape[0]

    @pl.when(i == 0)
    def _init():
        m_ref[...] = jnp.full(m_ref.shape, neg_inf, jnp.float32)
        z_ref[...] = jnp.zeros(z_ref.shape, jnp.float32)
        acc_ref[...] = jnp.zeros(acc_ref.shape, jnp.float32)

    xb16 = x_ref[...].astype(jnp.bfloat16)
    logits = jax.lax.dot_general(
        wa_ref[...].astype(jnp.bfloat16), xb16, (((1,), (1,)), ((), ())),
        preferred_element_type=jnp.float32) + ba_ref[...]
    t = jnp.tanh(logits)
    s = jnp.sum(t * cx_ref[...], axis=0, keepdims=True)

    bv = b_ref[0]
    seg = jax.lax.broadcasted_iota(jnp.int32, (num_seg, 1), 0)
    onehot = seg == bv
    oh16 = onehot.astype(jnp.bfloat16)

    m_blk = jnp.max(jnp.where(onehot, s, neg_inf), axis=1, keepdims=True)
    m_old = m_ref[...]
    m_new = jnp.maximum(m_old, m_blk).astype(jnp.bfloat16).astype(jnp.float32)
    scale = jnp.where(m_new == neg_inf, 1.0, jnp.exp(m_old - m_new))

    m_safe16 = jnp.where(m_new == neg_inf, 0.0, m_new).astype(jnp.bfloat16)
    m_row = jax.lax.dot_general(
        m_safe16, oh16, (((0,), (0,)), ((), ())),
        preferred_element_type=jnp.float32)
    e16 = jnp.exp(s - m_row).astype(jnp.bfloat16)
    w16 = oh16 * e16

    ones = jnp.ones((k, 1), jnp.bfloat16)
    z_blk = jax.lax.dot_general(
        w16, ones, (((1,), (0,)), ((), ())),
        preferred_element_type=jnp.float32)
    z_ref[...] = z_ref[...] * scale + z_blk
    acc_ref[...] = acc_ref[...] * scale + jax.lax.dot_general(
        w16, xb16, (((1,), (0,)), ((), ())),
        preferred_element_type=jnp.float32)
    m_ref[...] = m_new

    @pl.when(i == num_blocks - 1)
    def _finish():
        rep = acc_ref[...] / (z_ref[...] + 1e-8)
        out_ref[...] = jax.lax.dot_general(
            rep.astype(jnp.bfloat16), wo_ref[...].astype(jnp.bfloat16),
            (((1,), (1,)), ((), ())),
            preferred_element_type=jnp.float32) + bo_ref[...]


def _pick_block(n):
    for k in range(min(n, 10000), 7, -1):
        if n % k == 0 and k % 8 == 0:
            return k
    return None


@functools.partial(jax.jit, static_argnames=("num_segments", "interpret"))
def _pooled_attention(x, batch, W_att, b_att, context, W_out, b_out,
                      num_segments=_NUM_SEGMENTS, interpret=False):
    n, d = x.shape
    a = W_att.shape[0]
    k = _pick_block(n)
    if k is None:
        k = min(2048, 8 * ((n + 7) // 8))
        n_pad = ((n + k - 1) // k) * k
        x = jnp.pad(x, ((0, n_pad - n), (0, 0)))
        batch = jnp.pad(batch, (0, n_pad - n), constant_values=-1)
        n = n_pad
    g = n // k

    batch3 = batch.reshape(g, 1, k)
    ba2 = b_att.reshape(a, 1)
    cx2 = context.reshape(a, 1)
    bo2 = b_out.reshape(1, d)

    out = pl.pallas_call(
        _body,
        grid=(g,),
        in_specs=[
            pl.BlockSpec((k, d), lambda i: (i, 0)),
            pl.BlockSpec((1, 1, k), lambda i: (i, 0, 0)),
            pl.BlockSpec((a, d), lambda i: (0, 0)),
            pl.BlockSpec((a, 1), lambda i: (0, 0)),
            pl.BlockSpec((a, 1), lambda i: (0, 0)),
            pl.BlockSpec((d, d), lambda i: (0, 0)),
            pl.BlockSpec((1, d), lambda i: (0, 0)),
        ],
        out_specs=pl.BlockSpec((num_segments, d), lambda i: (0, 0)),
        out_shape=jax.ShapeDtypeStruct((num_segments, d), jnp.float32),
        scratch_shapes=[
            pltpu.VMEM((num_segments, 1), jnp.float32),
            pltpu.VMEM((num_segments, 1), jnp.float32),
            pltpu.VMEM((num_segments, d), jnp.float32),
        ],
        compiler_params=pltpu.CompilerParams(
            dimension_semantics=("arbitrary",)),
        interpret=interpret,
    )(x, batch3, W_att, ba2, cx2, W_out, bo2)
    return out


def kernel(x, batch, W_att, b_att, context, W_out, b_out):
    return _pooled_attention(x, batch, W_att, b_att, context, W_out, b_out)
